# baseline (speedup 1.0000x reference)
import jax
import jax.numpy as jnp
from jax import lax
from jax.experimental import pallas as pl
from jax.experimental.pallas import tpu as pltpu
from jax.experimental.pallas import tpu_sc as plsc

_MAX_REL_POS = 128
_S = 4096
_TAB_PAD = 144

_NC = 2
_NS = 16
_L = 16
_NW = _NC * _NS
_RB = _S // 8
_CT = _S // 128
_BPW = _RB // _NW
_NBT = 3


def _tec_body(table_hbm, out_hbm, table2d_v, table_v, buf0, buf1, sem0, sem1):
    wid = lax.axis_index("s") * _NC + lax.axis_index("c")
    c0 = jnp.maximum(0, jnp.minimum(_CT - _NBT, wid - 1))

    pltpu.sync_copy(table_hbm, table2d_v)
    zcol = jnp.zeros((_L,), jnp.int32)
    for k in range(_TAB_PAD // _L):
        rows = jnp.minimum(k * _L + lax.iota(jnp.int32, _L), _MAX_REL_POS)
        table_v[pl.ds(k * _L, _L)] = plsc.load_gather(table2d_v, (rows, zcol))

    fill = plsc.load_gather(
        table_v, (jnp.full((_L,), _MAX_REL_POS, jnp.int32),)
    )

    for buf in (buf0, buf1):

        def init_tile(c, carry, buf=buf):
            for r in range(8):
                for q in range(128 // _L):
                    buf[c, r, pl.ds(q * _L, _L)] = fill
            return carry

        lax.fori_loop(0, _CT, init_tile, 0)

    jv = [q * _L + lax.iota(jnp.int32, _L) for q in range(128 // _L)]

    def block(g, carry):
        for buf, sem, u in ((buf0, sem0, 0), (buf1, sem1, 1)):
            b = 2 * g + u
            blk_r = _BPW * wid + b

            @pl.when(g > 0)
            def _(buf=buf, sem=sem, blk_r=blk_r):
                pltpu.make_async_copy(buf, out_hbm.at[blk_r - 2], sem).wait()

            for t in range(_NBT):
                ct = c0 + t
                for r in range(8):
                    s = (8 * blk_r + r) - 128 * ct
                    for q in range(128 // _L):
                        d = jnp.minimum(jnp.abs(s - jv[q]), _MAX_REL_POS)
                        buf[ct, r, pl.ds(q * _L, _L)] = plsc.load_gather(
                            table_v, (d,)
                        )

            pltpu.make_async_copy(buf, out_hbm.at[blk_r], sem).start()
        return carry

    lax.fori_loop(0, _BPW // 2, block, 0)

    pltpu.make_async_copy(buf0, out_hbm.at[_BPW * wid + _BPW - 2], sem0).wait()
    pltpu.make_async_copy(buf1, out_hbm.at[_BPW * wid + _BPW - 1], sem1).wait()


@jax.jit
def _encode(table):
    kern = pl.kernel(
        _tec_body,
        out_type=jax.ShapeDtypeStruct((_RB, _CT, 8, 128), jnp.float32),
        mesh=plsc.VectorSubcoreMesh(
            core_axis_name="c", subcore_axis_name="s",
            num_cores=_NC, num_subcores=_NS,
        ),
        scratch_types=[
            pltpu.VMEM((_MAX_REL_POS + 1, 1), jnp.float32),
            pltpu.VMEM((_TAB_PAD,), jnp.float32),
            pltpu.VMEM((_CT, 8, 128), jnp.float32),
            pltpu.VMEM((_CT, 8, 128), jnp.float32),
            pltpu.SemaphoreType.DMA,
            pltpu.SemaphoreType.DMA,
        ],
        compiler_params=pltpu.CompilerParams(needs_layout_passes=False),
    )
    out4 = kern(table)
    return out4.transpose(0, 2, 1, 3).reshape(_S, _S)


def kernel(seq_len, rel_pos_emb):
    return _encode(rel_pos_emb)


# speedup vs baseline: 1.0357x; 1.0357x over previous
import jax
import jax.numpy as jnp
from jax import lax
from jax.experimental import pallas as pl
from jax.experimental.pallas import tpu as pltpu
from jax.experimental.pallas import tpu_sc as plsc

_MAX_REL_POS = 128
_S = 4096
_TAB_PAD = 144

_NC = 2
_NS = 16
_L = 16
_NW = _NC * _NS
_RB = _S // 8
_CT = _S // 128
_BPW = _RB // _NW
_NBT = 3


def _tec_body(table_hbm, out_hbm, table_v, buf0, buf1, sem0, sem1):
    wid = lax.axis_index("s") * _NC + lax.axis_index("c")
    c0 = jnp.maximum(0, jnp.minimum(_CT - _NBT, wid - 1))

    pltpu.sync_copy(table_hbm, table_v)

    fill = plsc.load_gather(
        table_v, (jnp.full((_L,), _MAX_REL_POS, jnp.int32),)
    )

    for buf in (buf0, buf1):

        def init_tile(c, carry, buf=buf):
            for r in range(8):
                for q in range(128 // _L):
                    buf[c, r, pl.ds(q * _L, _L)] = fill
            return carry

        lax.fori_loop(0, _CT, init_tile, 0)

    jv = [q * _L + lax.iota(jnp.int32, _L) for q in range(128 // _L)]

    def block(g, carry):
        for buf, sem, u in ((buf0, sem0, 0), (buf1, sem1, 1)):
            b = 2 * g + u
            blk_r = _BPW * wid + b

            @pl.when(g > 0)
            def _(buf=buf, sem=sem, blk_r=blk_r):
                pltpu.make_async_copy(buf, out_hbm.at[blk_r - 2], sem).wait()

            for t in range(_NBT):
                ct = c0 + t
                for r in range(8):
                    s = (8 * blk_r + r) - 128 * ct
                    for q in range(128 // _L):
                        d = jnp.minimum(jnp.abs(s - jv[q]), _MAX_REL_POS)
                        buf[ct, r, pl.ds(q * _L, _L)] = plsc.load_gather(
                            table_v, (d,)
                        )

            pltpu.make_async_copy(buf, out_hbm.at[blk_r], sem).start()
        return carry

    lax.fori_loop(0, _BPW // 2, block, 0)

    pltpu.make_async_copy(buf0, out_hbm.at[_BPW * wid + _BPW - 2], sem0).wait()
    pltpu.make_async_copy(buf1, out_hbm.at[_BPW * wid + _BPW - 1], sem1).wait()


@jax.jit
def _encode(table_pad):
    kern = pl.kernel(
        _tec_body,
        out_type=jax.ShapeDtypeStruct((_RB, _CT, 8, 128), jnp.float32),
        mesh=plsc.VectorSubcoreMesh(
            core_axis_name="c", subcore_axis_name="s",
            num_cores=_NC, num_subcores=_NS,
        ),
        scratch_types=[
            pltpu.VMEM((_TAB_PAD,), jnp.float32),
            pltpu.VMEM((_CT, 8, 128), jnp.float32),
            pltpu.VMEM((_CT, 8, 128), jnp.float32),
            pltpu.SemaphoreType.DMA,
            pltpu.SemaphoreType.DMA,
        ],
        compiler_params=pltpu.CompilerParams(needs_layout_passes=False),
    )
    out4 = kern(table_pad)
    return out4.transpose(0, 2, 1, 3).reshape(_S, _S)


def kernel(seq_len, rel_pos_emb):
    table = rel_pos_emb.reshape(-1)
    table_pad = jnp.zeros((_TAB_PAD,), jnp.float32).at[: table.shape[0]].set(table)
    return _encode(table_pad)
